# baseline (speedup 1.0000x reference)
import functools

import jax
import jax.numpy as jnp
from jax import lax
from jax.experimental import pallas as pl
from jax.experimental.pallas import tpu as pltpu
from jax.experimental.pallas import tpu_sc as plsc

EMBED = 32
SEQ = 200
BATCH = 4096
NC = 2
NS = 16
NW = NC * NS
BW = BATCH // NW
NN = 4
NBLK = SEQ // NN
ET = EMBED // 8

_mesh = plsc.VectorSubcoreMesh(core_axis_name="c", subcore_axis_name="s")


@functools.partial(
    pl.kernel,
    mesh=_mesh,
    out_type=jax.ShapeDtypeStruct((SEQ, ET, NW, 8, BW), jnp.float32),
    scratch_types=[
        pltpu.VMEM((BW, SEQ), jnp.int32),
        pltpu.VMEM((SEQ, BW), jnp.int32),
        pltpu.VMEM((NN, BW, EMBED), jnp.float32),
        pltpu.VMEM((NN, BW, EMBED), jnp.float32),
        pltpu.VMEM((ET, 8, BW), jnp.float32),
        pltpu.VMEM((ET, 8, BW), jnp.float32),
        pltpu.VMEM((SEQ, EMBED), jnp.float32),
        pltpu.SemaphoreType.DMA,
        pltpu.SemaphoreType.DMA,
        pltpu.SemaphoreType.DMA,
        pltpu.SemaphoreType.DMA,
    ],
    compiler_params=pltpu.CompilerParams(
        use_tc_tiling_on_sc=False, needs_layout_passes=False),
)
def _embed_lookup(x_hbm, tok_hbm, pos_hbm, out_hbm,
                  slab, slabT, rows0, rows1, t0, t1, pos_v,
                  s_g0, s_g1, s_t0, s_t1):
    rows = (rows0, rows1)
    tcol = (t0, t1)
    s_g = (s_g0, s_g1)
    s_t = (s_t0, s_t1)

    w = lax.axis_index("s") * NC + lax.axis_index("c")
    base = w * BW
    pltpu.sync_copy(pos_hbm, pos_v)
    pltpu.sync_copy(x_hbm.at[pl.ds(base, BW)], slab)

    iota = jnp.arange(16, dtype=jnp.int32)
    cvecs = [c * 16 + iota for c in range(8)]

    def tr_body(n, carry):
        for c in range(8):
            v = plsc.load_gather(
                slab, [cvecs[c], jnp.full((16,), n, jnp.int32)])
            slabT[n, pl.ds(c * 16, 16)] = v
        return carry

    lax.fori_loop(0, SEQ, tr_body, 0, unroll=4)

    def start_gathers(blk, rb):
        for j in range(NN):
            n = blk * NN + j
            pltpu.async_copy(tok_hbm.at[slabT.at[n]], rows[rb].at[j], s_g[rb])

    def wait_gathers(blk, rb):
        for j in range(NN):
            n = blk * NN + j
            pltpu.make_async_copy(
                tok_hbm.at[slabT.at[n]], rows[rb].at[j], s_g[rb]).wait()

    def start_twrite(n, tb):
        pltpu.async_copy(tcol[tb], out_hbm.at[n, :, w], s_t[tb])

    def wait_twrite(tb):
        pltpu.make_async_copy(tcol[tb], out_hbm.at[0, :, w], s_t[tb]).wait()

    def transpose_add(rb, j, tb, n):
        jv = jnp.full((16,), j, jnp.int32)
        rv = rows[rb]
        tv = tcol[tb]

        def e_body(e, carry):
            nv = jnp.full((16,), n, jnp.int32)
            ev = jnp.full((16,), e, jnp.int32)
            pb = plsc.load_gather(pos_v, [nv, ev])
            a = e >> 3
            bs = e & 7
            for cb in range(8):
                vals = plsc.load_gather(rv, [jv, cvecs[cb], ev])
                tv[a, bs, pl.ds(cb * 16, 16)] = vals + pb
            return carry

        lax.fori_loop(0, EMBED, e_body, 0, unroll=8)

    start_gathers(0, 0)
    start_gathers(1, 1)

    def outer(t2, carry):
        for rb in range(2):
            blk = 2 * t2 + rb
            wait_gathers(blk, rb)
            for j in range(NN):
                n = blk * NN + j
                tb = j & 1

                @pl.when(n >= 2)
                def _():
                    wait_twrite(tb)

                transpose_add(rb, j, tb, n)
                start_twrite(n, tb)

            @pl.when(blk + 2 < NBLK)
            def _():
                start_gathers(blk + 2, rb)
        return carry

    lax.fori_loop(0, NBLK // 2, outer, 0)
    wait_twrite(0)
    wait_twrite(1)


def kernel(x, tok_table, pos_table):
    out5 = _embed_lookup(x.astype(jnp.int32), tok_table, pos_table)
    return jnp.transpose(out5, (2, 4, 0, 1, 3)).reshape(BATCH, SEQ, EMBED)


# speedup vs baseline: 1.4672x; 1.4672x over previous
import functools

import jax
import jax.numpy as jnp
from jax import lax
from jax.experimental import pallas as pl
from jax.experimental.pallas import tpu as pltpu
from jax.experimental.pallas import tpu_sc as plsc

EMBED = 32
SEQ = 200
BATCH = 4096
NC = 2
NS = 16
NW = NC * NS
BW = BATCH // NW
NN = 4
NBLK = SEQ // NN
ET = EMBED // 8

_mesh = plsc.VectorSubcoreMesh(core_axis_name="c", subcore_axis_name="s")


@functools.partial(
    pl.kernel,
    mesh=_mesh,
    out_type=jax.ShapeDtypeStruct((SEQ, ET, NW, 8, BW), jnp.float32),
    scratch_types=[
        pltpu.VMEM((BW, SEQ), jnp.int32),
        pltpu.VMEM((SEQ, BW), jnp.int32),
        pltpu.VMEM((NN, BW, EMBED), jnp.float32),
        pltpu.VMEM((NN, BW, EMBED), jnp.float32),
        pltpu.VMEM((ET, 8, BW), jnp.float32),
        pltpu.VMEM((ET, 8, BW), jnp.float32),
        pltpu.VMEM((SEQ, EMBED), jnp.float32),
        pltpu.SemaphoreType.DMA,
        pltpu.SemaphoreType.DMA,
        pltpu.SemaphoreType.DMA,
        pltpu.SemaphoreType.DMA,
    ],
    compiler_params=pltpu.CompilerParams(
        use_tc_tiling_on_sc=False, needs_layout_passes=False),
)
def _embed_lookup(x_hbm, tok_hbm, pos_hbm, out_hbm,
                  slab, slabT, rows0, rows1, t0, t1, pos_v,
                  s_g0, s_g1, s_t0, s_t1):
    rows = (rows0, rows1)
    tcol = (t0, t1)
    s_g = (s_g0, s_g1)
    s_t = (s_t0, s_t1)

    w = lax.axis_index("s") * NC + lax.axis_index("c")
    base = w * BW
    pltpu.sync_copy(pos_hbm, pos_v)
    pltpu.sync_copy(x_hbm.at[pl.ds(base, BW)], slab)

    iota = jnp.arange(16, dtype=jnp.int32)
    cvecs = [c * 16 + iota for c in range(8)]

    def tr_body(n, carry):
        for c in range(8):
            v = plsc.load_gather(
                slab, [cvecs[c], jnp.full((16,), n, jnp.int32)])
            slabT[n, pl.ds(c * 16, 16)] = v
        return carry

    lax.fori_loop(0, SEQ, tr_body, 0, unroll=4)

    def start_gathers(blk, rb):
        for j in range(NN):
            n = blk * NN + j
            pltpu.async_copy(tok_hbm.at[slabT.at[n]], rows[rb].at[j], s_g[rb])

    def wait_gathers(blk, rb):
        for j in range(NN):
            n = blk * NN + j
            pltpu.make_async_copy(
                tok_hbm.at[slabT.at[n]], rows[rb].at[j], s_g[rb]).wait()

    def start_twrite(n, tb):
        pltpu.async_copy(tcol[tb], out_hbm.at[n, :, w], s_t[tb])

    def wait_twrite(tb):
        pltpu.make_async_copy(tcol[tb], out_hbm.at[0, :, w], s_t[tb]).wait()

    def transpose_add(rb, j, tb, n):
        jv = jnp.full((16,), j, jnp.int32)
        nv = jnp.full((16,), n, jnp.int32)
        rv = rows[rb]
        tv = tcol[tb]

        def k_body(k, carry):
            ek = (iota + k) & 15
            for e0 in (0, 16):
                ev = ek + e0
                pb = plsc.load_gather(pos_v, [nv, ev])
                av = ev >> 3
                bv = ev & 7
                for cb in range(8):
                    vals = plsc.load_gather(rv, [jv, cvecs[cb], ev])
                    plsc.store_scatter(tv, [av, bv, cvecs[cb]], vals + pb)
            return carry

        lax.fori_loop(0, 16, k_body, 0, unroll=2)

    start_gathers(0, 0)
    start_gathers(1, 1)

    def outer(t2, carry):
        for rb in range(2):
            blk = 2 * t2 + rb
            wait_gathers(blk, rb)
            for j in range(NN):
                n = blk * NN + j
                tb = j & 1

                @pl.when(n >= 2)
                def _():
                    wait_twrite(tb)

                transpose_add(rb, j, tb, n)
                start_twrite(n, tb)

            @pl.when(blk + 2 < NBLK)
            def _():
                start_gathers(blk + 2, rb)
        return carry

    lax.fori_loop(0, NBLK // 2, outer, 0)
    wait_twrite(0)
    wait_twrite(1)


def kernel(x, tok_table, pos_table):
    out5 = _embed_lookup(x.astype(jnp.int32), tok_table, pos_table)
    return jnp.transpose(out5, (2, 4, 0, 1, 3)).reshape(BATCH, SEQ, EMBED)


# speedup vs baseline: 1.5671x; 1.0681x over previous
import functools

import jax
import jax.numpy as jnp
from jax import lax
from jax.experimental import pallas as pl
from jax.experimental.pallas import tpu as pltpu
from jax.experimental.pallas import tpu_sc as plsc

EMBED = 32
SEQ = 200
BATCH = 4096
NC = 2
NS = 16
NW = NC * NS
BW = BATCH // NW
NN = 4
NBLK = SEQ // NN
ET = EMBED // 8

_mesh = plsc.VectorSubcoreMesh(core_axis_name="c", subcore_axis_name="s")


@functools.partial(
    pl.kernel,
    mesh=_mesh,
    out_type=jax.ShapeDtypeStruct((SEQ, ET, NW, 8, BW), jnp.float32),
    scratch_types=[
        pltpu.VMEM((BW, SEQ), jnp.int32),
        pltpu.VMEM((SEQ, BW), jnp.int32),
        pltpu.VMEM((NN, BW, EMBED), jnp.float32),
        pltpu.VMEM((NN, BW, EMBED), jnp.float32),
        pltpu.VMEM((ET, 8, BW), jnp.float32),
        pltpu.VMEM((ET, 8, BW), jnp.float32),
        pltpu.VMEM((SEQ, EMBED), jnp.float32),
        pltpu.SemaphoreType.DMA,
        pltpu.SemaphoreType.DMA,
        pltpu.SemaphoreType.DMA,
        pltpu.SemaphoreType.DMA,
    ],
    compiler_params=pltpu.CompilerParams(
        use_tc_tiling_on_sc=False, needs_layout_passes=False),
)
def _embed_lookup(x_hbm, tok_hbm, pos_hbm, out_hbm,
                  slab, slabT, rows0, rows1, t0, t1, pos_v,
                  s_g0, s_g1, s_t0, s_t1):
    rows = (rows0, rows1)
    tcol = (t0, t1)
    s_g = (s_g0, s_g1)
    s_t = (s_t0, s_t1)

    w = lax.axis_index("s") * NC + lax.axis_index("c")
    base = w * BW
    pltpu.sync_copy(pos_hbm, pos_v)
    pltpu.sync_copy(x_hbm.at[pl.ds(base, BW)], slab)

    iota = jnp.arange(16, dtype=jnp.int32)
    cvecs = [c * 16 + iota for c in range(8)]

    def tr_body(n, carry):
        for c in range(8):
            v = plsc.load_gather(
                slab, [cvecs[c], jnp.full((16,), n, jnp.int32)])
            slabT[n, pl.ds(c * 16, 16)] = v
        return carry

    lax.fori_loop(0, SEQ, tr_body, 0, unroll=4)

    def start_gathers(blk, rb):
        for j in range(NN):
            n = blk * NN + j
            pltpu.async_copy(tok_hbm.at[slabT.at[n]], rows[rb].at[j], s_g[rb])

    def wait_gathers(blk, rb):
        for j in range(NN):
            n = blk * NN + j
            pltpu.make_async_copy(
                tok_hbm.at[slabT.at[n]], rows[rb].at[j], s_g[rb]).wait()

    def start_twrite(n, tb):
        pltpu.async_copy(tcol[tb], out_hbm.at[n, :, w], s_t[tb])

    def wait_twrite(tb):
        pltpu.make_async_copy(tcol[tb], out_hbm.at[0, :, w], s_t[tb]).wait()

    def transpose_add(rb, j, tb, n):
        jv = jnp.full((16,), j, jnp.int32)
        nv = jnp.full((16,), n, jnp.int32)
        rv = rows[rb]
        tv = tcol[tb]

        def k_body(k, carry):
            ek = (iota + k) & 15
            for e0 in (0, 16):
                ev = ek + e0
                pb = plsc.load_gather(pos_v, [nv, ev])
                av = ev >> 3
                bv = ev & 7
                for cb in range(8):
                    vals = plsc.load_gather(rv, [jv, cvecs[cb], ev])
                    plsc.store_scatter(tv, [av, bv, cvecs[cb]], vals + pb)
            return carry

        lax.fori_loop(0, 16, k_body, 0, unroll=4)

    start_gathers(0, 0)
    start_gathers(1, 1)

    def outer(t2, carry):
        for rb in range(2):
            blk = 2 * t2 + rb
            wait_gathers(blk, rb)
            for j in range(NN):
                n = blk * NN + j
                tb = j & 1

                @pl.when(n >= 2)
                def _():
                    wait_twrite(tb)

                transpose_add(rb, j, tb, n)
                start_twrite(n, tb)

            @pl.when(blk + 2 < NBLK)
            def _():
                start_gathers(blk + 2, rb)
        return carry

    lax.fori_loop(0, NBLK // 2, outer, 0)
    wait_twrite(0)
    wait_twrite(1)


def kernel(x, tok_table, pos_table):
    out5 = _embed_lookup(x.astype(jnp.int32), tok_table, pos_table)
    return jnp.transpose(out5, (2, 4, 0, 1, 3)).reshape(BATCH, SEQ, EMBED)
